# baseline (speedup 1.0000x reference)
import functools

import jax
import jax.numpy as jnp
from jax import lax
from jax.experimental import pallas as pl
from jax.experimental.pallas import tpu as pltpu
from jax.experimental.pallas import tpu_sc as plsc

_PPH = 128
_NCU = 16
_NQ = 4
_GR = 96
_CR = 8
_IR = 32
_UPS = 4


def _sc_gather(points_lin, gidx_lin, sample_idx, BH, N, S):
    PPH = _PPH
    mesh = plsc.VectorSubcoreMesh(core_axis_name="c", subcore_axis_name="s",
                                  num_cores=2, num_subcores=16)

    @functools.partial(
        pl.kernel,
        out_type=(
            jax.ShapeDtypeStruct((BH, _NCU * _GR, 128), jnp.float32),
            jax.ShapeDtypeStruct((BH, _NCU * _CR, 128), jnp.float32),
        ),
        mesh=mesh,
        compiler_params=pltpu.CompilerParams(needs_layout_passes=False),
        scratch_types=(
            pltpu.VMEM((N * 3 // 128, 128), jnp.float32),
            pltpu.VMEM((_IR, 128), jnp.int32),
            pltpu.VMEM((PPH,), jnp.int32),
            pltpu.VMEM((_GR, 128), jnp.float32),
            pltpu.VMEM((_CR, 128), jnp.float32),
        ),
    )
    def sc_kern(pts_hbm, gidx_hbm, sidx_hbm, grp_hbm, ctr_hbm,
                pts_v, gidx_v, sidx_v, grp_v, ctr_v):
        cid = lax.axis_index("c")
        sid = lax.axis_index("s")
        wid = cid * 16 + sid
        b = wid // _NCU
        u = wid % _NCU
        pltpu.sync_copy(pts_hbm.at[b], pts_v)
        pltpu.sync_copy(gidx_hbm.at[b, pl.ds(u * _IR, _IR)], gidx_v)
        pltpu.sync_copy(sidx_hbm.at[b, pl.ds(u * PPH, PPH)], sidx_v)

        iota = lax.broadcasted_iota(jnp.int32, (16,), 0)

        def coord(i3):
            return plsc.load_gather(pts_v, [i3 >> 7, i3 & 127])

        @plsc.parallel_loop(0, PPH // 16)
        def block_loop(pb):
            base = pb * 16
            ci3 = sidx_v[pl.ds(base, 16)] * 3
            for c in range(3):
                ctr_v[c, pl.ds(base, 16)] = coord(ci3 + c)
            pv = (iota + base) * S
            for s in range(S):
                gf = pv + s
                gi3 = plsc.load_gather(gidx_v, [gf >> 7, gf & 127]) * 3
                for c in range(3):
                    grp_v[c * S + s, pl.ds(base, 16)] = coord(gi3 + c)

        pltpu.sync_copy(grp_v, grp_hbm.at[b, pl.ds(u * _GR, _GR)])
        pltpu.sync_copy(ctr_v, ctr_hbm.at[b, pl.ds(u * _CR, _CR)])

    return sc_kern(points_lin, gidx_lin, sample_idx)


def _tc_mlp_body(qo, par_ref, rel_ref, ctr_ref, out_ref):
    b = pl.program_id(0) + qo

    def w1(k, j):
        return par_ref[56 + k * 8 + j]

    def w2(k, j):
        return par_ref[80 + k * 8 + j]

    def w3(k, j):
        return par_ref[144 + k * 16 + j]

    def w4(k, j):
        return par_ref[272 + k * 3 + j]

    q0 = par_ref[b * 4 + 0]
    q1 = par_ref[b * 4 + 1]
    q2 = par_ref[b * 4 + 2]
    q3 = par_ref[b * 4 + 3]
    s_ = q0 * q0 + q1 * q1 + q2 * q2 + q3 * q3 + 1e-10
    inv_norm = 1.0 / (jnp.sqrt(s_) + 1e-10)
    a0 = q0 * inv_norm
    a1 = q1 * inv_norm
    a2 = q2 * inv_norm
    a3 = q3 * inv_norm
    n2 = a0 * a0 + a1 * a1 + a2 * a2 + a3 * a3 + 1e-10
    c0 = a0 / n2
    c1 = -a1 / n2
    c2 = -a2 / n2
    c3 = -a3 / n2
    tx = par_ref[32 + b * 3 + 0]
    ty = par_ref[32 + b * 3 + 1]
    tz = par_ref[32 + b * 3 + 2]

    for v in range(_UPS):
        ro = v * _GR
        co = v * _CR
        px = ctr_ref[0, co + 0]
        py = ctr_ref[0, co + 1]
        pz = ctr_ref[0, co + 2]
        px8 = jnp.broadcast_to(px[None], (8, 128))
        py8 = jnp.broadcast_to(py[None], (8, 128))
        pz8 = jnp.broadcast_to(pz[None], (8, 128))
        rx = rel_ref[0, ro + 0:ro + 32].reshape(4, 8, 128) - px8[None]
        ry = rel_ref[0, ro + 32:ro + 64].reshape(4, 8, 128) - py8[None]
        rz = rel_ref[0, ro + 64:ro + 96].reshape(4, 8, 128) - pz8[None]
        h1 = [jnp.maximum(rx * w1(0, j) + ry * w1(1, j) + rz * w1(2, j),
                          0.0) for j in range(8)]
        h2 = []
        for j in range(8):
            acc = h1[0] * w2(0, j)
            for k in range(1, 8):
                acc = acc + h1[k] * w2(k, j)
            h2.append(jnp.maximum(acc, 0.0))
        m = []
        for j in range(16):
            acc = h2[0] * w3(0, j)
            for k in range(1, 8):
                acc = acc + h2[k] * w3(k, j)
            mf = jnp.max(jnp.maximum(acc, 0.0), axis=0)
            m.append(jnp.max(mf, axis=0))
        flow = []
        for c in range(3):
            acc = m[0] * w4(0, c)
            for k in range(1, 16):
                acc = acc + m[k] * w4(k, c)
            flow.append(acc)
        r0 = -a1 * px - a2 * py - a3 * pz
        r1 = a0 * px - a2 * pz - a3 * py
        r2 = a0 * py - a1 * pz - a3 * px
        r3 = a0 * pz - a1 * py - a2 * px
        w_1 = r0 * c1 - r1 * c0 - r2 * c3 - r3 * c2
        w_2 = r0 * c2 - r1 * c3 - r2 * c0 - r3 * c1
        w_3 = r0 * c3 - r1 * c2 - r2 * c1 - r3 * c0
        out_ref[0, co + 0] = w_1 + tx + flow[0]
        out_ref[0, co + 1] = w_2 + ty + flow[1]
        out_ref[0, co + 2] = w_3 + tz + flow[2]


def _tc_stage(qo, params, rel, ctr, BH, S):
    return pl.pallas_call(
        functools.partial(_tc_mlp_body, qo),
        grid=(BH, _NCU // _UPS),
        in_specs=[
            pl.BlockSpec(memory_space=pltpu.SMEM),
            pl.BlockSpec((1, _UPS * _GR, 128), lambda b, c: (b, c, 0)),
            pl.BlockSpec((1, _UPS * _CR, 128), lambda b, c: (b, c, 0)),
        ],
        out_specs=pl.BlockSpec((1, _UPS * _CR, 128), lambda b, c: (b, c, 0)),
        out_shape=jax.ShapeDtypeStruct((BH, _NCU * _CR, 128), jnp.float32),
    )(params, rel, ctr)


def kernel(points, q, t, sample_idx, group_idx, W1, W2, W3, W4):
    B, N, _ = points.shape
    _, P = sample_idx.shape
    S = group_idx.shape[2]
    BH = B // _NQ

    params = jnp.concatenate([
        q.reshape(-1), t.reshape(-1), W1.reshape(-1),
        W2.reshape(-1), W3.reshape(-1), W4.reshape(-1)])
    outs = []
    for h in range(_NQ):
        sl = slice(h * BH, (h + 1) * BH)
        pts_lin = points[sl].reshape(BH, N * 3 // 128, 128)
        gidx_lin = group_idx[sl].astype(jnp.int32).reshape(
            BH, P * S // 128, 128)
        sidx = sample_idx[sl].astype(jnp.int32)
        rel, ctr = _sc_gather(pts_lin, gidx_lin, sidx, BH, N, S)
        outs.append(_tc_stage(h * BH, params, rel, ctr, BH, S))

    out_t = jnp.concatenate(outs, axis=0)
    return (out_t.reshape(B, _NCU, _CR, 128)[:, :, 0:3]
            .transpose(0, 1, 3, 2)
            .reshape(B, P, 3))


# speedup vs baseline: 1.1419x; 1.1419x over previous
import functools

import jax
import jax.numpy as jnp
from jax import lax
from jax.experimental import pallas as pl
from jax.experimental.pallas import tpu as pltpu
from jax.experimental.pallas import tpu_sc as plsc

_PPH = 256
_NC2 = 8


def _sc_gather(points_lin, gidx_lin, sample_idx, BH, N, S):
    PPH = _PPH
    GR = 3 * S * PPH // 128
    CR = 8
    IR = PPH * S // 128
    mesh = plsc.VectorSubcoreMesh(core_axis_name="c", subcore_axis_name="s",
                                  num_cores=2, num_subcores=16)

    @functools.partial(
        pl.kernel,
        out_type=(
            jax.ShapeDtypeStruct((BH, _NC2 * GR, 128), jnp.float32),
            jax.ShapeDtypeStruct((BH, _NC2 * CR, 128), jnp.float32),
        ),
        mesh=mesh,
        compiler_params=pltpu.CompilerParams(needs_layout_passes=False),
        scratch_types=(
            pltpu.VMEM((N * 3 // 128, 128), jnp.float32),
            pltpu.VMEM((IR, 128), jnp.int32),
            pltpu.VMEM((PPH,), jnp.int32),
            pltpu.VMEM((GR, 128), jnp.float32),
            pltpu.VMEM((CR, 128), jnp.float32),
        ),
    )
    def sc_kern(pts_hbm, gidx_hbm, sidx_hbm, grp_hbm, ctr_hbm,
                pts_v, gidx_v, sidx_v, grp_v, ctr_v):
        cid = lax.axis_index("c")
        sid = lax.axis_index("s")
        wid = cid * 16 + sid
        b = wid // _NC2
        ch = wid % _NC2
        pltpu.sync_copy(pts_hbm.at[b], pts_v)
        pltpu.sync_copy(gidx_hbm.at[b, pl.ds(ch * IR, IR)], gidx_v)
        pltpu.sync_copy(sidx_hbm.at[b, pl.ds(ch * PPH, PPH)], sidx_v)

        iota = lax.broadcasted_iota(jnp.int32, (16,), 0)
        zero = jnp.zeros((16,), jnp.int32)

        def coord(i3):
            return plsc.load_gather(pts_v, [i3 >> 7, i3 & 127])

        @plsc.parallel_loop(0, PPH // 16)
        def block_loop(pb):
            base = pb * 16
            ci3 = sidx_v[pl.ds(base, 16)] * 3
            for c in range(3):
                f = c * PPH + base
                plsc.store_scatter(
                    ctr_v, [zero + (f >> 7), iota + (f & 127)],
                    coord(ci3 + c))
            pv = (iota + base) * S
            for s in range(S):
                gf = pv + s
                gi3 = plsc.load_gather(gidx_v, [gf >> 7, gf & 127]) * 3
                for c in range(3):
                    f = c * S * PPH + s * PPH + base
                    plsc.store_scatter(
                        grp_v, [zero + (f >> 7), iota + (f & 127)],
                        coord(gi3 + c))

        pltpu.sync_copy(grp_v, grp_hbm.at[b, pl.ds(ch * GR, GR)])
        pltpu.sync_copy(ctr_v, ctr_hbm.at[b, pl.ds(ch * CR, CR)])

    return sc_kern(points_lin, gidx_lin, sample_idx)


def _tc_mlp_body(qo, par_ref, rel_ref, ctr_ref, out_ref):
    b = pl.program_id(0) + qo

    def w1(k, j):
        return par_ref[56 + k * 8 + j]

    def w2(k, j):
        return par_ref[80 + k * 8 + j]

    def w3(k, j):
        return par_ref[144 + k * 16 + j]

    def w4(k, j):
        return par_ref[272 + k * 3 + j]

    q0 = par_ref[b * 4 + 0]
    q1 = par_ref[b * 4 + 1]
    q2 = par_ref[b * 4 + 2]
    q3 = par_ref[b * 4 + 3]
    s_ = q0 * q0 + q1 * q1 + q2 * q2 + q3 * q3 + 1e-10
    inv_norm = 1.0 / (jnp.sqrt(s_) + 1e-10)
    a0 = q0 * inv_norm
    a1 = q1 * inv_norm
    a2 = q2 * inv_norm
    a3 = q3 * inv_norm
    n2 = a0 * a0 + a1 * a1 + a2 * a2 + a3 * a3 + 1e-10
    c0 = a0 / n2
    c1 = -a1 / n2
    c2 = -a2 / n2
    c3 = -a3 / n2
    tx = par_ref[32 + b * 3 + 0]
    ty = par_ref[32 + b * 3 + 1]
    tz = par_ref[32 + b * 3 + 2]

    for u in range(2):
        ro = u * 192
        co = u * 8
        px = ctr_ref[0, co + 0:co + 2]
        py = ctr_ref[0, co + 2:co + 4]
        pz = ctr_ref[0, co + 4:co + 6]
        px8 = jnp.concatenate([px, px, px, px], axis=0)
        py8 = jnp.concatenate([py, py, py, py], axis=0)
        pz8 = jnp.concatenate([pz, pz, pz, pz], axis=0)
        rx = rel_ref[0, ro + 0:ro + 64].reshape(8, 8, 128) - px8[None]
        ry = rel_ref[0, ro + 64:ro + 128].reshape(8, 8, 128) - py8[None]
        rz = rel_ref[0, ro + 128:ro + 192].reshape(8, 8, 128) - pz8[None]
        h1 = [jnp.maximum(rx * w1(0, j) + ry * w1(1, j) + rz * w1(2, j),
                          0.0) for j in range(8)]
        h2 = []
        for j in range(8):
            acc = h1[0] * w2(0, j)
            for k in range(1, 8):
                acc = acc + h1[k] * w2(k, j)
            h2.append(jnp.maximum(acc, 0.0))
        m = []
        for j in range(16):
            acc = h2[0] * w3(0, j)
            for k in range(1, 8):
                acc = acc + h2[k] * w3(k, j)
            mf = jnp.max(jnp.maximum(acc, 0.0), axis=0)
            m.append(jnp.max(mf.reshape(4, 2, 128), axis=0))
        flow = []
        for c in range(3):
            acc = m[0] * w4(0, c)
            for k in range(1, 16):
                acc = acc + m[k] * w4(k, c)
            flow.append(acc)
        r0 = -a1 * px - a2 * py - a3 * pz
        r1 = a0 * px - a2 * pz - a3 * py
        r2 = a0 * py - a1 * pz - a3 * px
        r3 = a0 * pz - a1 * py - a2 * px
        w_1 = r0 * c1 - r1 * c0 - r2 * c3 - r3 * c2
        w_2 = r0 * c2 - r1 * c3 - r2 * c0 - r3 * c1
        w_3 = r0 * c3 - r1 * c2 - r2 * c1 - r3 * c0
        out_ref[0, co + 0:co + 2] = w_1 + tx + flow[0]
        out_ref[0, co + 2:co + 4] = w_2 + ty + flow[1]
        out_ref[0, co + 4:co + 6] = w_3 + tz + flow[2]


def _tc_stage(qo, params, rel, ctr, BH, S):
    GR = 3 * S * _PPH // 128
    CR = 8
    return pl.pallas_call(
        functools.partial(_tc_mlp_body, qo),
        grid=(BH, _NC2 // 2),
        in_specs=[
            pl.BlockSpec(memory_space=pltpu.SMEM),
            pl.BlockSpec((1, 2 * GR, 128), lambda b, c: (b, c, 0)),
            pl.BlockSpec((1, 2 * CR, 128), lambda b, c: (b, c, 0)),
        ],
        out_specs=pl.BlockSpec((1, 2 * CR, 128), lambda b, c: (b, c, 0)),
        out_shape=jax.ShapeDtypeStruct((BH, _NC2 * CR, 128), jnp.float32),
    )(params, rel, ctr)


def kernel(points, q, t, sample_idx, group_idx, W1, W2, W3, W4):
    B, N, _ = points.shape
    _, P = sample_idx.shape
    S = group_idx.shape[2]
    BH = B // 2

    params = jnp.concatenate([
        q.reshape(-1), t.reshape(-1), W1.reshape(-1),
        W2.reshape(-1), W3.reshape(-1), W4.reshape(-1)])
    outs = []
    for h in range(2):
        sl = slice(h * BH, (h + 1) * BH)
        pts_lin = points[sl].reshape(BH, N * 3 // 128, 128)
        gidx_lin = group_idx[sl].astype(jnp.int32).reshape(
            BH, P * S // 128, 128)
        sidx = sample_idx[sl].astype(jnp.int32)
        rel, ctr = _sc_gather(pts_lin, gidx_lin, sidx, BH, N, S)
        outs.append(_tc_stage(h * BH, params, rel, ctr, BH, S))

    out_t = jnp.concatenate(outs, axis=0)
    return (out_t.reshape(B, _NC2, 4, 2, 128)[:, :, 0:3]
            .transpose(0, 1, 3, 4, 2)
            .reshape(B, P, 3))


# speedup vs baseline: 1.1609x; 1.0166x over previous
import functools

import jax
import jax.numpy as jnp
from jax import lax
from jax.experimental import pallas as pl
from jax.experimental.pallas import tpu as pltpu
from jax.experimental.pallas import tpu_sc as plsc

_PPH = 256
_NC2 = 8


def _sc_gather(points_lin, gidx_lin, sample_idx, BH, N, S):
    PPH = _PPH
    GR = 3 * S * PPH // 128
    CR = 8
    IR = PPH * S // 128
    mesh = plsc.VectorSubcoreMesh(core_axis_name="c", subcore_axis_name="s",
                                  num_cores=2, num_subcores=16)

    @functools.partial(
        pl.kernel,
        out_type=(
            jax.ShapeDtypeStruct((BH, _NC2 * GR, 128), jnp.float32),
            jax.ShapeDtypeStruct((BH, _NC2 * CR, 128), jnp.float32),
        ),
        mesh=mesh,
        compiler_params=pltpu.CompilerParams(needs_layout_passes=False),
        scratch_types=(
            pltpu.VMEM((N * 3 // 128, 128), jnp.float32),
            pltpu.VMEM((IR, 128), jnp.int32),
            pltpu.VMEM((PPH,), jnp.int32),
            pltpu.VMEM((GR, 128), jnp.float32),
            pltpu.VMEM((CR, 128), jnp.float32),
        ),
    )
    def sc_kern(pts_hbm, gidx_hbm, sidx_hbm, grp_hbm, ctr_hbm,
                pts_v, gidx_v, sidx_v, grp_v, ctr_v):
        cid = lax.axis_index("c")
        sid = lax.axis_index("s")
        wid = cid * 16 + sid
        b = wid // _NC2
        ch = wid % _NC2
        pltpu.sync_copy(pts_hbm.at[b], pts_v)
        pltpu.sync_copy(gidx_hbm.at[b, pl.ds(ch * IR, IR)], gidx_v)
        pltpu.sync_copy(sidx_hbm.at[b, pl.ds(ch * PPH, PPH)], sidx_v)

        iota = lax.broadcasted_iota(jnp.int32, (16,), 0)
        zero = jnp.zeros((16,), jnp.int32)

        def coord(i3):
            return plsc.load_gather(pts_v, [i3 >> 7, i3 & 127])

        @plsc.parallel_loop(0, PPH // 16)
        def block_loop(pb):
            base = pb * 16
            ci3 = sidx_v[pl.ds(base, 16)] * 3
            for c in range(3):
                f = c * PPH + base
                plsc.store_scatter(
                    ctr_v, [zero + (f >> 7), iota + (f & 127)],
                    coord(ci3 + c))
            pv = (iota + base) * S
            for s in range(S):
                gf = pv + s
                gi3 = plsc.load_gather(gidx_v, [gf >> 7, gf & 127]) * 3
                for c in range(3):
                    f = c * S * PPH + s * PPH + base
                    plsc.store_scatter(
                        grp_v, [zero + (f >> 7), iota + (f & 127)],
                        coord(gi3 + c))

        pltpu.sync_copy(grp_v, grp_hbm.at[b, pl.ds(ch * GR, GR)])
        pltpu.sync_copy(ctr_v, ctr_hbm.at[b, pl.ds(ch * CR, CR)])

    return sc_kern(points_lin, gidx_lin, sample_idx)


def _tc_mlp_body(qo, par_ref, rel_ref, ctr_ref, out_ref):
    b = pl.program_id(0) + qo

    def w1(k, j):
        return par_ref[56 + k * 8 + j]

    def w2(k, j):
        return par_ref[80 + k * 8 + j]

    def w3(k, j):
        return par_ref[144 + k * 16 + j]

    def w4(k, j):
        return par_ref[272 + k * 3 + j]

    q0 = par_ref[b * 4 + 0]
    q1 = par_ref[b * 4 + 1]
    q2 = par_ref[b * 4 + 2]
    q3 = par_ref[b * 4 + 3]
    s_ = q0 * q0 + q1 * q1 + q2 * q2 + q3 * q3 + 1e-10
    inv_norm = 1.0 / (jnp.sqrt(s_) + 1e-10)
    a0 = q0 * inv_norm
    a1 = q1 * inv_norm
    a2 = q2 * inv_norm
    a3 = q3 * inv_norm
    n2 = a0 * a0 + a1 * a1 + a2 * a2 + a3 * a3 + 1e-10
    c0 = a0 / n2
    c1 = -a1 / n2
    c2 = -a2 / n2
    c3 = -a3 / n2
    tx = par_ref[32 + b * 3 + 0]
    ty = par_ref[32 + b * 3 + 1]
    tz = par_ref[32 + b * 3 + 2]

    for u in range(4):
        ro = u * 192
        co = u * 8
        px = ctr_ref[0, co + 0:co + 2]
        py = ctr_ref[0, co + 2:co + 4]
        pz = ctr_ref[0, co + 4:co + 6]
        px8 = jnp.concatenate([px, px, px, px], axis=0)
        py8 = jnp.concatenate([py, py, py, py], axis=0)
        pz8 = jnp.concatenate([pz, pz, pz, pz], axis=0)
        rx = rel_ref[0, ro + 0:ro + 64].reshape(8, 8, 128) - px8[None]
        ry = rel_ref[0, ro + 64:ro + 128].reshape(8, 8, 128) - py8[None]
        rz = rel_ref[0, ro + 128:ro + 192].reshape(8, 8, 128) - pz8[None]
        h1 = [jnp.maximum(rx * w1(0, j) + ry * w1(1, j) + rz * w1(2, j),
                          0.0) for j in range(8)]
        h2 = []
        for j in range(8):
            acc = h1[0] * w2(0, j)
            for k in range(1, 8):
                acc = acc + h1[k] * w2(k, j)
            h2.append(jnp.maximum(acc, 0.0))
        m = []
        for j in range(16):
            acc = h2[0] * w3(0, j)
            for k in range(1, 8):
                acc = acc + h2[k] * w3(k, j)
            mf = jnp.max(jnp.maximum(acc, 0.0), axis=0)
            m.append(jnp.max(mf.reshape(4, 2, 128), axis=0))
        flow = []
        for c in range(3):
            acc = m[0] * w4(0, c)
            for k in range(1, 16):
                acc = acc + m[k] * w4(k, c)
            flow.append(acc)
        r0 = -a1 * px - a2 * py - a3 * pz
        r1 = a0 * px - a2 * pz - a3 * py
        r2 = a0 * py - a1 * pz - a3 * px
        r3 = a0 * pz - a1 * py - a2 * px
        w_1 = r0 * c1 - r1 * c0 - r2 * c3 - r3 * c2
        w_2 = r0 * c2 - r1 * c3 - r2 * c0 - r3 * c1
        w_3 = r0 * c3 - r1 * c2 - r2 * c1 - r3 * c0
        out_ref[0, co + 0:co + 2] = w_1 + tx + flow[0]
        out_ref[0, co + 2:co + 4] = w_2 + ty + flow[1]
        out_ref[0, co + 4:co + 6] = w_3 + tz + flow[2]


def _tc_stage(qo, params, rel, ctr, BH, S):
    GR = 3 * S * _PPH // 128
    CR = 8
    return pl.pallas_call(
        functools.partial(_tc_mlp_body, qo),
        grid=(BH, _NC2 // 4),
        in_specs=[
            pl.BlockSpec(memory_space=pltpu.SMEM),
            pl.BlockSpec((1, 4 * GR, 128), lambda b, c: (b, c, 0)),
            pl.BlockSpec((1, 4 * CR, 128), lambda b, c: (b, c, 0)),
        ],
        out_specs=pl.BlockSpec((1, 4 * CR, 128), lambda b, c: (b, c, 0)),
        out_shape=jax.ShapeDtypeStruct((BH, _NC2 * CR, 128), jnp.float32),
    )(params, rel, ctr)


def kernel(points, q, t, sample_idx, group_idx, W1, W2, W3, W4):
    B, N, _ = points.shape
    _, P = sample_idx.shape
    S = group_idx.shape[2]
    BH = B // 2

    params = jnp.concatenate([
        q.reshape(-1), t.reshape(-1), W1.reshape(-1),
        W2.reshape(-1), W3.reshape(-1), W4.reshape(-1)])
    outs = []
    for h in range(2):
        sl = slice(h * BH, (h + 1) * BH)
        pts_lin = points[sl].reshape(BH, N * 3 // 128, 128)
        gidx_lin = group_idx[sl].astype(jnp.int32).reshape(
            BH, P * S // 128, 128)
        sidx = sample_idx[sl].astype(jnp.int32)
        rel, ctr = _sc_gather(pts_lin, gidx_lin, sidx, BH, N, S)
        outs.append(_tc_stage(h * BH, params, rel, ctr, BH, S))

    out_t = jnp.concatenate(outs, axis=0)
    return (out_t.reshape(B, _NC2, 4, 2, 128)[:, :, 0:3]
            .transpose(0, 1, 3, 4, 2)
            .reshape(B, P, 3))
